# baseline (speedup 1.0000x reference)
import functools

import jax
import jax.numpy as jnp
from jax import lax
from jax.experimental import pallas as pl
from jax.experimental.pallas import tpu as pltpu
from jax.experimental.pallas import tpu_sc as plsc

N = 100000
D = 128
EMB = 128
C = 40
B = 16384
S1 = 5
S2 = 10
ALPHA = 0.2

NC = 2
NS = 16
NW = NC * NS

_MESH = functools.partial(
    plsc.VectorSubcoreMesh, core_axis_name="c", subcore_axis_name="s",
    num_cores=NC, num_subcores=NS)


def _wid():
    return lax.axis_index("s") * NC + lax.axis_index("c")


PA = 160
CH_A = N // PA
IDXA = PA * S1
A_SPLIT = (125, 255, 245)


def _make_l1(chunk_lo, chunk_n):
    pn = chunk_n * PA
    iters = -(-chunk_n // NW)

    def body(neigh_hbm, feat_hbm, out_hbm, idx_v, rows_v, sem_i, sem_g):
        w = _wid()

        def chunk(it, carry):
            c = chunk_lo + w + it * NW

            @pl.when(c < chunk_lo + chunk_n)
            def _():
                base = c * PA
                rbase = (c - chunk_lo) * PA
                icps = [
                    pltpu.async_copy(
                        neigh_hbm.at[pl.ds(j * N + base, PA)],
                        idx_v.at[pl.ds(j * PA, PA)], sem_i)
                    for j in range(S1)
                ]
                for d_ in icps:
                    d_.wait()
                pltpu.async_copy(feat_hbm.at[idx_v], rows_v, sem_g).wait()
                ocps = [
                    pltpu.async_copy(
                        rows_v.at[pl.ds(j * PA, PA)],
                        out_hbm.at[pl.ds(j * pn + rbase, PA)], sem_i)
                    for j in range(S1)
                ]
                for d_ in ocps:
                    d_.wait()

            return carry

        lax.fori_loop(0, iters, chunk, 0)

    return pl.kernel(
        body,
        out_type=jax.ShapeDtypeStruct((S1 * pn, D), jnp.float32),
        mesh=_MESH(),
        scratch_types=[
            pltpu.VMEM((IDXA,), jnp.int32),
            pltpu.VMEM((IDXA, D), jnp.float32),
            pltpu.SemaphoreType.DMA,
            pltpu.SemaphoreType.DMA,
        ],
    )


_A_LO = [sum(A_SPLIT[:i]) for i in range(len(A_SPLIT))]
_l1_parts = [_make_l1(lo, n) for lo, n in zip(_A_LO, A_SPLIT)]


QC = 32
CH_C = B // QC
HIDX = QC * 16
GH = 4
SGH = HIDX // GH
C_SPLIT = (96, 224, 192)


def _make_l2(chunk_lo, chunk_n):
    iters = chunk_n // NW

    def body(nodes_hbm, tblg_hbm, h1_hbm, out_hbm,
             nv, gidx, grp_v, hidx, rows_v, sem):
        w = _wid()
        lanes = lax.iota(jnp.int32, 16)

        def chunk(it, carry):
            c = chunk_lo + w * iters + it
            base = c * QC
            rbase = (c - chunk_lo) * QC
            pltpu.sync_copy(nodes_hbm.at[pl.ds(base, QC)], nv)
            for t in range(QC // 16):
                nv16 = nv[pl.ds(t * 16, 16)]
                gidx[pl.ds(t * 16, 16)] = jnp.right_shift(nv16, 3)
            pltpu.async_copy(tblg_hbm.at[gidx], grp_v, sem).wait()
            for t in range(QC // 16):
                nv16 = nv[pl.ds(t * 16, 16)]
                for q in range(16):
                    i = t * 16 + q
                    node = nv16[q]
                    vals = grp_v[i, pl.ds(jnp.bitwise_and(node, 7) * 16, 16)]
                    hidx[pl.ds(i * 16, 16)] = vals
            gathers = [
                pltpu.async_copy(
                    h1_hbm.at[hidx.at[pl.ds(k * SGH, SGH)]],
                    rows_v.at[pl.ds(k * SGH, SGH)], sem)
                for k in range(GH)
            ]
            for d_ in gathers:
                d_.wait()
            pltpu.sync_copy(rows_v, out_hbm.at[pl.ds(rbase * 16, HIDX)])
            return carry

        lax.fori_loop(0, iters, chunk, 0)

    return pl.kernel(
        body,
        out_type=jax.ShapeDtypeStruct((chunk_n * QC * 16, EMB), jnp.float32),
        mesh=_MESH(),
        scratch_types=[
            pltpu.VMEM((QC,), jnp.int32),
            pltpu.VMEM((QC,), jnp.int32),
            pltpu.VMEM((QC, 128), jnp.int32),
            pltpu.VMEM((HIDX,), jnp.int32),
            pltpu.VMEM((HIDX, EMB), jnp.float32),
            pltpu.SemaphoreType.DMA,
        ],
    )


_C_LO = [sum(C_SPLIT[:i]) for i in range(len(C_SPLIT))]
_l2_parts = [_make_l2(lo, n) for lo, n in zip(_C_LO, C_SPLIT)]


BM1 = 800


def _mm1_body(g_ref, f_ref, w_ref, o_ref):
    s = f_ref[...]
    for j in range(S1):
        s = s + g_ref[j]
    y = jnp.dot(s, w_ref[...],
                preferred_element_type=jnp.float32) * (1.0 / (S1 + 1))
    o_ref[...] = jnp.where(y >= 0, y, ALPHA * y)


def _h1_tc_part(g_part, feats, w1, node_lo, pn):
    blk_lo = node_lo // BM1
    return pl.pallas_call(
        _mm1_body,
        grid=(pn // BM1,),
        in_specs=[
            pl.BlockSpec((S1, BM1, D), lambda i: (0, i, 0)),
            pl.BlockSpec((BM1, D), lambda i: (i + blk_lo, 0)),
            pl.BlockSpec((D, EMB), lambda i: (0, 0)),
        ],
        out_specs=pl.BlockSpec((BM1, EMB), lambda i: (i, 0)),
        out_shape=jax.ShapeDtypeStruct((pn, EMB), jnp.float32),
    )(g_part, feats, w1)


BM2 = 256


def _mm2_body(r_ref, w_ref, cw_ref, o_ref):
    s = r_ref[:, 0, :]
    for k in range(1, S2 + 1):
        s = s + r_ref[:, k, :]
    y = jnp.dot(s, w_ref[...],
                preferred_element_type=jnp.float32) * (1.0 / (S2 + 1))
    h = jnp.where(y >= 0, y, ALPHA * y)
    o_ref[...] = jnp.dot(h, cw_ref[...], preferred_element_type=jnp.float32)


def _head_tc_part(rows_part, w2, cw_t, nb):
    return pl.pallas_call(
        _mm2_body,
        grid=(nb // BM2,),
        in_specs=[
            pl.BlockSpec((BM2, 16, EMB), lambda i: (i, 0, 0)),
            pl.BlockSpec((EMB, EMB), lambda i: (0, 0)),
            pl.BlockSpec((EMB, C), lambda i: (0, 0)),
        ],
        out_specs=pl.BlockSpec((BM2, C), lambda i: (i, 0)),
        out_shape=jax.ShapeDtypeStruct((nb, C), jnp.float32),
    )(rows_part, w2, cw_t)


def kernel(nodes, neigh_l1, neigh_l2, features, W1, W2, class_weight):
    neigh_t = neigh_l1.T.reshape(S1 * N)
    self6 = jnp.broadcast_to(jnp.arange(N, dtype=jnp.int32)[:, None], (N, 6))
    tblg = jnp.concatenate([neigh_l2, self6], axis=1).reshape(N // 8, 128)
    cw_t = class_weight.T

    slabs = []
    for part, (lo, n) in enumerate(zip(_A_LO, A_SPLIT)):
        g = _l1_parts[part](neigh_t, features).reshape(S1, n * PA, D)
        slabs.append(_h1_tc_part(g, features, W1, lo * PA, n * PA))
    h1 = jnp.concatenate(slabs, axis=0)

    scores = []
    for part, (lo, n) in enumerate(zip(_C_LO, C_SPLIT)):
        rows = _l2_parts[part](nodes, tblg, h1)
        scores.append(
            _head_tc_part(rows.reshape(n * QC, 16, EMB), W2, cw_t, n * QC))
    return jnp.concatenate(scores, axis=0)


# speedup vs baseline: 1.0265x; 1.0265x over previous
import functools

import jax
import jax.numpy as jnp
from jax import lax
from jax.experimental import pallas as pl
from jax.experimental.pallas import tpu as pltpu
from jax.experimental.pallas import tpu_sc as plsc

N = 100000
D = 128
EMB = 128
C = 40
B = 16384
S1 = 5
S2 = 10
ALPHA = 0.2

NC = 2
NS = 16
NW = NC * NS

_MESH = functools.partial(
    plsc.VectorSubcoreMesh, core_axis_name="c", subcore_axis_name="s",
    num_cores=NC, num_subcores=NS)


def _wid():
    return lax.axis_index("s") * NC + lax.axis_index("c")


PA = 160
CH_A = N // PA
IDXA = PA * S1
A_SPLIT = (125, 255, 245)


def _make_l1(chunk_lo, chunk_n):
    pn = chunk_n * PA
    iters = -(-chunk_n // NW)

    def body(neigh_hbm, feat_hbm, out_hbm, idx_v, rows_v, sem_i, sem_g):
        w = _wid()

        def chunk(it, carry):
            c = chunk_lo + w + it * NW

            @pl.when(c < chunk_lo + chunk_n)
            def _():
                base = c * PA
                rbase = (c - chunk_lo) * PA
                icps = [
                    pltpu.async_copy(
                        neigh_hbm.at[pl.ds(j * N + base, PA)],
                        idx_v.at[pl.ds(j * PA, PA)], sem_i)
                    for j in range(S1)
                ]
                for d_ in icps:
                    d_.wait()
                pltpu.async_copy(feat_hbm.at[idx_v], rows_v, sem_g).wait()
                ocps = [
                    pltpu.async_copy(
                        rows_v.at[pl.ds(j * PA, PA)],
                        out_hbm.at[pl.ds(j * pn + rbase, PA)], sem_i)
                    for j in range(S1)
                ]
                for d_ in ocps:
                    d_.wait()

            return carry

        lax.fori_loop(0, iters, chunk, 0)

    return pl.kernel(
        body,
        out_type=jax.ShapeDtypeStruct((S1 * pn, D), jnp.float32),
        mesh=_MESH(),
        scratch_types=[
            pltpu.VMEM((IDXA,), jnp.int32),
            pltpu.VMEM((IDXA, D), jnp.float32),
            pltpu.SemaphoreType.DMA,
            pltpu.SemaphoreType.DMA,
        ],
    )


_A_LO = [sum(A_SPLIT[:i]) for i in range(len(A_SPLIT))]
_l1_parts = [_make_l1(lo, n) for lo, n in zip(_A_LO, A_SPLIT)]


QC = 32
CH_C = B // QC
HIDX = QC * 16
GH = 4
SGH = HIDX // GH
C_SPLIT = (96, 224, 192)


def _make_l2(chunk_lo, chunk_n):
    iters = chunk_n // NW

    def body(nodes_hbm, tblg_hbm, h1_hbm, out_hbm,
             nv, gidx, grp_v, hidx, rows_v, sem):
        w = _wid()
        lanes = lax.iota(jnp.int32, 16)

        def chunk(it, carry):
            c = chunk_lo + w * iters + it
            base = c * QC
            rbase = (c - chunk_lo) * QC
            pltpu.sync_copy(nodes_hbm.at[pl.ds(base, QC)], nv)
            for t in range(QC // 16):
                nv16 = nv[pl.ds(t * 16, 16)]
                gidx[pl.ds(t * 16, 16)] = jnp.right_shift(nv16, 3)
            pltpu.async_copy(tblg_hbm.at[gidx], grp_v, sem).wait()
            for t in range(QC // 16):
                nv16 = nv[pl.ds(t * 16, 16)]
                for q in range(16):
                    i = t * 16 + q
                    node = nv16[q]
                    vals = grp_v[i, pl.ds(jnp.bitwise_and(node, 7) * 16, 16)]
                    hidx[pl.ds(i * 16, 16)] = vals
            gathers = [
                pltpu.async_copy(
                    h1_hbm.at[hidx.at[pl.ds(k * SGH, SGH)]],
                    rows_v.at[pl.ds(k * SGH, SGH)], sem)
                for k in range(GH)
            ]
            for d_ in gathers:
                d_.wait()
            pltpu.sync_copy(rows_v, out_hbm.at[pl.ds(rbase * 16, HIDX)])
            return carry

        lax.fori_loop(0, iters, chunk, 0)

    return pl.kernel(
        body,
        out_type=jax.ShapeDtypeStruct((chunk_n * QC * 16, EMB), jnp.float32),
        mesh=_MESH(),
        scratch_types=[
            pltpu.VMEM((QC,), jnp.int32),
            pltpu.VMEM((QC,), jnp.int32),
            pltpu.VMEM((QC, 128), jnp.int32),
            pltpu.VMEM((HIDX,), jnp.int32),
            pltpu.VMEM((HIDX, EMB), jnp.float32),
            pltpu.SemaphoreType.DMA,
        ],
    )


_C_LO = [sum(C_SPLIT[:i]) for i in range(len(C_SPLIT))]
_l2_parts = [_make_l2(lo, n) for lo, n in zip(_C_LO, C_SPLIT)]


BM1 = 800


def _mm1_body(h_ref, g_ref, f_ref, w_ref, o_ref):
    s = f_ref[...]
    for j in range(S1):
        s = s + g_ref[j]
    y = jnp.dot(s, w_ref[...],
                preferred_element_type=jnp.float32) * (1.0 / (S1 + 1))
    o_ref[...] = jnp.where(y >= 0, y, ALPHA * y)


def _h1_tc_part(h1buf, g_part, feats, w1, node_lo, pn):
    blk_lo = node_lo // BM1
    return pl.pallas_call(
        _mm1_body,
        grid=(pn // BM1,),
        in_specs=[
            pl.BlockSpec((BM1, EMB), lambda i: (i + blk_lo, 0)),
            pl.BlockSpec((S1, BM1, D), lambda i: (0, i, 0)),
            pl.BlockSpec((BM1, D), lambda i: (i + blk_lo, 0)),
            pl.BlockSpec((D, EMB), lambda i: (0, 0)),
        ],
        out_specs=pl.BlockSpec((BM1, EMB), lambda i: (i + blk_lo, 0)),
        out_shape=jax.ShapeDtypeStruct((N, EMB), jnp.float32),
        input_output_aliases={0: 0},
    )(h1buf, g_part, feats, w1)


BM2 = 256


def _mm2_body(r_ref, w_ref, cw_ref, o_ref):
    s = r_ref[:, 0, :]
    for k in range(1, S2 + 1):
        s = s + r_ref[:, k, :]
    y = jnp.dot(s, w_ref[...],
                preferred_element_type=jnp.float32) * (1.0 / (S2 + 1))
    h = jnp.where(y >= 0, y, ALPHA * y)
    o_ref[...] = jnp.dot(h, cw_ref[...], preferred_element_type=jnp.float32)


def _head_tc_part(rows_part, w2, cw_t, nb):
    return pl.pallas_call(
        _mm2_body,
        grid=(nb // BM2,),
        in_specs=[
            pl.BlockSpec((BM2, 16, EMB), lambda i: (i, 0, 0)),
            pl.BlockSpec((EMB, EMB), lambda i: (0, 0)),
            pl.BlockSpec((EMB, C), lambda i: (0, 0)),
        ],
        out_specs=pl.BlockSpec((BM2, C), lambda i: (i, 0)),
        out_shape=jax.ShapeDtypeStruct((nb, C), jnp.float32),
    )(rows_part, w2, cw_t)


def kernel(nodes, neigh_l1, neigh_l2, features, W1, W2, class_weight):
    neigh_t = neigh_l1.T.reshape(S1 * N)
    self6 = jnp.broadcast_to(jnp.arange(N, dtype=jnp.int32)[:, None], (N, 6))
    tblg = jnp.concatenate([neigh_l2, self6], axis=1).reshape(N // 8, 128)
    cw_t = class_weight.T

    h1 = jnp.zeros((N, EMB), jnp.float32)
    for part, (lo, n) in enumerate(zip(_A_LO, A_SPLIT)):
        g = _l1_parts[part](neigh_t, features).reshape(S1, n * PA, D)
        h1 = _h1_tc_part(h1, g, features, W1, lo * PA, n * PA)

    scores = []
    for part, (lo, n) in enumerate(zip(_C_LO, C_SPLIT)):
        rows = _l2_parts[part](nodes, tblg, h1)
        scores.append(
            _head_tc_part(rows.reshape(n * QC, 16, EMB), W2, cw_t, n * QC))
    return jnp.concatenate(scores, axis=0)


# speedup vs baseline: 1.0267x; 1.0002x over previous
import functools

import jax
import jax.numpy as jnp
from jax import lax
from jax.experimental import pallas as pl
from jax.experimental.pallas import tpu as pltpu
from jax.experimental.pallas import tpu_sc as plsc

N = 100000
D = 128
EMB = 128
C = 40
B = 16384
S1 = 5
S2 = 10
ALPHA = 0.2

NC = 2
NS = 16
NW = NC * NS

_MESH = functools.partial(
    plsc.VectorSubcoreMesh, core_axis_name="c", subcore_axis_name="s",
    num_cores=NC, num_subcores=NS)


def _wid():
    return lax.axis_index("s") * NC + lax.axis_index("c")


PA = 160
CH_A = N // PA
IDXA = PA * S1
A_SPLIT = (125, 255, 245)


def _make_l1(chunk_lo, chunk_n):
    pn = chunk_n * PA
    iters = -(-chunk_n // NW)

    def body(neigh_hbm, feat_hbm, out_hbm, idx_v, rows_v, sem_i, sem_g):
        w = _wid()

        def chunk(it, carry):
            c = chunk_lo + w + it * NW

            @pl.when(c < chunk_lo + chunk_n)
            def _():
                base = c * PA
                rbase = (c - chunk_lo) * PA
                icps = [
                    pltpu.async_copy(
                        neigh_hbm.at[pl.ds(j * N + base, PA)],
                        idx_v.at[pl.ds(j * PA, PA)], sem_i)
                    for j in range(S1)
                ]
                for d_ in icps:
                    d_.wait()
                gcs = [
                    pltpu.async_copy(
                        feat_hbm.at[idx_v.at[pl.ds(h * (IDXA // 2), IDXA // 2)]],
                        rows_v.at[pl.ds(h * (IDXA // 2), IDXA // 2)], sem_g)
                    for h in range(2)
                ]
                for d_ in gcs:
                    d_.wait()
                ocps = [
                    pltpu.async_copy(
                        rows_v.at[pl.ds(j * PA, PA)],
                        out_hbm.at[pl.ds(j * pn + rbase, PA)], sem_i)
                    for j in range(S1)
                ]
                for d_ in ocps:
                    d_.wait()

            return carry

        lax.fori_loop(0, iters, chunk, 0)

    return pl.kernel(
        body,
        out_type=jax.ShapeDtypeStruct((S1 * pn, D), jnp.float32),
        mesh=_MESH(),
        scratch_types=[
            pltpu.VMEM((IDXA,), jnp.int32),
            pltpu.VMEM((IDXA, D), jnp.float32),
            pltpu.SemaphoreType.DMA,
            pltpu.SemaphoreType.DMA,
        ],
    )


_A_LO = [sum(A_SPLIT[:i]) for i in range(len(A_SPLIT))]
_l1_parts = [_make_l1(lo, n) for lo, n in zip(_A_LO, A_SPLIT)]


QC = 32
CH_C = B // QC
HIDX = QC * 16
GH = 4
SGH = HIDX // GH
C_SPLIT = (96, 224, 192)


def _make_l2(chunk_lo, chunk_n):
    iters = chunk_n // NW

    def body(nodes_hbm, tblg_hbm, h1_hbm, out_hbm,
             nv, gidx, grp_v, hidx, rows_v, sem):
        w = _wid()
        lanes = lax.iota(jnp.int32, 16)

        def chunk(it, carry):
            c = chunk_lo + w * iters + it
            base = c * QC
            rbase = (c - chunk_lo) * QC
            pltpu.sync_copy(nodes_hbm.at[pl.ds(base, QC)], nv)
            for t in range(QC // 16):
                nv16 = nv[pl.ds(t * 16, 16)]
                gidx[pl.ds(t * 16, 16)] = jnp.right_shift(nv16, 3)
            pltpu.async_copy(tblg_hbm.at[gidx], grp_v, sem).wait()
            for t in range(QC // 16):
                nv16 = nv[pl.ds(t * 16, 16)]
                for q in range(16):
                    i = t * 16 + q
                    node = nv16[q]
                    vals = grp_v[i, pl.ds(jnp.bitwise_and(node, 7) * 16, 16)]
                    hidx[pl.ds(i * 16, 16)] = vals
            gathers = [
                pltpu.async_copy(
                    h1_hbm.at[hidx.at[pl.ds(k * SGH, SGH)]],
                    rows_v.at[pl.ds(k * SGH, SGH)], sem)
                for k in range(GH)
            ]
            for d_ in gathers:
                d_.wait()
            pltpu.sync_copy(rows_v, out_hbm.at[pl.ds(rbase * 16, HIDX)])
            return carry

        lax.fori_loop(0, iters, chunk, 0)

    return pl.kernel(
        body,
        out_type=jax.ShapeDtypeStruct((chunk_n * QC * 16, EMB), jnp.float32),
        mesh=_MESH(),
        scratch_types=[
            pltpu.VMEM((QC,), jnp.int32),
            pltpu.VMEM((QC,), jnp.int32),
            pltpu.VMEM((QC, 128), jnp.int32),
            pltpu.VMEM((HIDX,), jnp.int32),
            pltpu.VMEM((HIDX, EMB), jnp.float32),
            pltpu.SemaphoreType.DMA,
        ],
    )


_C_LO = [sum(C_SPLIT[:i]) for i in range(len(C_SPLIT))]
_l2_parts = [_make_l2(lo, n) for lo, n in zip(_C_LO, C_SPLIT)]


BM1 = 800


def _mm1_body(h_ref, g_ref, f_ref, w_ref, o_ref):
    s = f_ref[...]
    for j in range(S1):
        s = s + g_ref[j]
    y = jnp.dot(s, w_ref[...],
                preferred_element_type=jnp.float32) * (1.0 / (S1 + 1))
    o_ref[...] = jnp.where(y >= 0, y, ALPHA * y)


def _h1_tc_part(h1buf, g_part, feats, w1, node_lo, pn):
    blk_lo = node_lo // BM1
    return pl.pallas_call(
        _mm1_body,
        grid=(pn // BM1,),
        in_specs=[
            pl.BlockSpec((BM1, EMB), lambda i: (i + blk_lo, 0)),
            pl.BlockSpec((S1, BM1, D), lambda i: (0, i, 0)),
            pl.BlockSpec((BM1, D), lambda i: (i + blk_lo, 0)),
            pl.BlockSpec((D, EMB), lambda i: (0, 0)),
        ],
        out_specs=pl.BlockSpec((BM1, EMB), lambda i: (i + blk_lo, 0)),
        out_shape=jax.ShapeDtypeStruct((N, EMB), jnp.float32),
        input_output_aliases={0: 0},
    )(h1buf, g_part, feats, w1)


BM2 = 256


def _mm2_body(r_ref, w_ref, cw_ref, o_ref):
    s = r_ref[:, 0, :]
    for k in range(1, S2 + 1):
        s = s + r_ref[:, k, :]
    y = jnp.dot(s, w_ref[...],
                preferred_element_type=jnp.float32) * (1.0 / (S2 + 1))
    h = jnp.where(y >= 0, y, ALPHA * y)
    o_ref[...] = jnp.dot(h, cw_ref[...], preferred_element_type=jnp.float32)


def _head_tc_part(rows_part, w2, cw_t, nb):
    return pl.pallas_call(
        _mm2_body,
        grid=(nb // BM2,),
        in_specs=[
            pl.BlockSpec((BM2, 16, EMB), lambda i: (i, 0, 0)),
            pl.BlockSpec((EMB, EMB), lambda i: (0, 0)),
            pl.BlockSpec((EMB, C), lambda i: (0, 0)),
        ],
        out_specs=pl.BlockSpec((BM2, C), lambda i: (i, 0)),
        out_shape=jax.ShapeDtypeStruct((nb, C), jnp.float32),
    )(rows_part, w2, cw_t)


def kernel(nodes, neigh_l1, neigh_l2, features, W1, W2, class_weight):
    neigh_t = neigh_l1.T.reshape(S1 * N)
    self6 = jnp.broadcast_to(jnp.arange(N, dtype=jnp.int32)[:, None], (N, 6))
    tblg = jnp.concatenate([neigh_l2, self6], axis=1).reshape(N // 8, 128)
    cw_t = class_weight.T

    h1 = jnp.zeros((N, EMB), jnp.float32)
    for part, (lo, n) in enumerate(zip(_A_LO, A_SPLIT)):
        g = _l1_parts[part](neigh_t, features).reshape(S1, n * PA, D)
        h1 = _h1_tc_part(h1, g, features, W1, lo * PA, n * PA)

    scores = []
    for part, (lo, n) in enumerate(zip(_C_LO, C_SPLIT)):
        rows = _l2_parts[part](nodes, tblg, h1)
        scores.append(
            _head_tc_part(rows.reshape(n * QC, 16, EMB), W2, cw_t, n * QC))
    return jnp.concatenate(scores, axis=0)
